# Initial kernel scaffold; baseline (speedup 1.0000x reference)
import functools

import jax
import jax.numpy as jnp
from jax import lax
from jax.experimental import pallas as pl
from jax.experimental.pallas import tpu as pltpu
from jax.experimental.pallas import tpu_sc as plsc

LANES = 16
BATCH = 128
NCORES = 2
NSUB = 16
NTILES = NCORES * NSUB


def _hist_kernel(NP, rows_per_tile):
    mesh = plsc.VectorSubcoreMesh(core_axis_name="c", subcore_axis_name="s")
    ST = NP // NSUB

    @functools.partial(
        pl.kernel,
        out_type=jax.ShapeDtypeStruct((NCORES, NP, LANES), jnp.float32),
        mesh=mesh,
        scratch_types=[
            pltpu.VMEM((rows_per_tile, BATCH), jnp.int32),
            pltpu.VMEM((BATCH, LANES), jnp.float32),
            pltpu.VMEM_SHARED((NP, LANES), jnp.float32),
        ],
    )
    def k(dst_hbm, out_hbm, dst_v, ones_v, hist_sh):
        c = lax.axis_index("c")
        s = lax.axis_index("s")
        base = s * ST

        @pl.loop(0, BATCH)
        def _(i):
            ones_v[i, :] = jnp.zeros((LANES,), jnp.float32)

        for r0 in range(0, ST, BATCH):
            sz = min(BATCH, ST - r0)
            pltpu.sync_copy(ones_v.at[pl.ds(0, sz)],
                            hist_sh.at[pl.ds(base + r0, sz)])

        @pl.loop(0, BATCH)
        def _(i):
            ones_v[i, :] = jnp.ones((LANES,), jnp.float32)

        plsc.subcore_barrier()

        row0 = (c * NSUB + s) * rows_per_tile
        pltpu.sync_copy(dst_hbm.at[pl.ds(row0, rows_per_tile)], dst_v)

        @pl.loop(0, rows_per_tile)
        def _(j):
            pltpu.sync_copy(ones_v, hist_sh.at[dst_v.at[j]], add=True)

        plsc.subcore_barrier()
        pltpu.sync_copy(hist_sh.at[pl.ds(base, ST)],
                        out_hbm.at[c].at[pl.ds(base, ST)])

    return k


def _scatter_kernel(NP, D, rows_per_tile, col_split):
    mesh = plsc.VectorSubcoreMesh(core_axis_name="c", subcore_axis_name="s")
    ST = NP // NSUB

    @functools.partial(
        pl.kernel,
        out_type=jax.ShapeDtypeStruct((NCORES, NP, D), jnp.float32),
        mesh=mesh,
        scratch_types=[
            pltpu.VMEM((rows_per_tile, BATCH), jnp.int32),
            pltpu.VMEM((rows_per_tile, BATCH), jnp.int32),
            pltpu.VMEM((BATCH, D), jnp.float32),
            pltpu.VMEM_SHARED((NP, D), jnp.float32),
        ],
    )
    def k(y_hbm, src_hbm, dst_hbm, out_hbm, src_v, dst_v, rows_v, agg_sh):
        c = lax.axis_index("c")
        s = lax.axis_index("s")
        base = s * ST

        @pl.loop(0, BATCH)
        def _(i):
            @pl.loop(0, D // LANES)
            def _(kk):
                rows_v[i, pl.ds(kk * LANES, LANES)] = (
                    jnp.zeros((LANES,), jnp.float32))

        for r0 in range(0, ST, BATCH):
            sz = min(BATCH, ST - r0)
            pltpu.sync_copy(rows_v.at[pl.ds(0, sz)],
                            agg_sh.at[pl.ds(base + r0, sz)])
        plsc.subcore_barrier()

        if col_split:
            row0 = s * rows_per_tile
            table = y_hbm.at[c]
        else:
            row0 = (c * NSUB + s) * rows_per_tile
            table = y_hbm

        pltpu.sync_copy(src_hbm.at[pl.ds(row0, rows_per_tile)], src_v)
        pltpu.sync_copy(dst_hbm.at[pl.ds(row0, rows_per_tile)], dst_v)

        @pl.loop(0, rows_per_tile)
        def _(j):
            pltpu.sync_copy(table.at[src_v.at[j]], rows_v)
            pltpu.sync_copy(rows_v, agg_sh.at[dst_v.at[j]], add=True)

        plsc.subcore_barrier()
        pltpu.sync_copy(agg_sh.at[pl.ds(base, ST)],
                        out_hbm.at[c].at[pl.ds(base, ST)])

    return k


def _mm_body(x_ref, w_ref, o_ref):
    o_ref[...] = jnp.dot(x_ref[...], w_ref[...],
                         preferred_element_type=jnp.float32)


def _scale_body(xw_ref, hist_ref, y_ref, dinv_ref):
    deg = (jnp.sum(hist_ref[0], axis=-1)
           + jnp.sum(hist_ref[1], axis=-1)) * (1.0 / LANES) + 1.0
    dinv = lax.rsqrt(deg)
    dinv_ref[...] = dinv
    y = xw_ref[...] * dinv[:, None]
    h = y.shape[1] // 2
    y_ref[0] = y[:, :h]
    y_ref[1] = y[:, h:]


def _mid_body(agg_ref, y1_ref, dinv_ref, b1_ref, w2_ref, y2_ref):
    dinv = dinv_ref[...][:, None]
    h = w2_ref.shape[0] // 2
    h0 = jnp.maximum((agg_ref[0] + y1_ref[0]) * dinv + b1_ref[:h][None, :],
                     0.0)
    h1 = jnp.maximum((agg_ref[1] + y1_ref[1]) * dinv + b1_ref[h:][None, :],
                     0.0)
    y2 = jnp.dot(h0, w2_ref[:h], preferred_element_type=jnp.float32)
    y2 = y2 + jnp.dot(h1, w2_ref[h:], preferred_element_type=jnp.float32)
    y2_ref[...] = y2 * dinv


def _fin_body(agg_ref, y2_ref, dinv_ref, b2_ref, o_ref):
    dinv = dinv_ref[...][:, None]
    o_ref[...] = ((agg_ref[0] + agg_ref[1] + y2_ref[...]) * dinv
                  + b2_ref[None, :])


def kernel(x, edge_index, W1, b1, W2, b2):
    N, F = x.shape
    HID = W1.shape[1]
    C = W2.shape[1]
    E = edge_index.shape[1]

    NP = ((N + 1 + NSUB - 1) // NSUB) * NSUB
    EP = ((E + BATCH * NTILES - 1) // (BATCH * NTILES)) * (BATCH * NTILES)
    pad = EP - E
    nrows = EP // BATCH

    srcp = jnp.concatenate(
        [edge_index[0], jnp.full((pad,), N, jnp.int32)]).reshape(nrows, BATCH)
    dstp = jnp.concatenate(
        [edge_index[1], jnp.full((pad,), N, jnp.int32)]).reshape(nrows, BATCH)
    xp = jnp.concatenate([x, jnp.zeros((NP - N, F), x.dtype)])

    f32 = jnp.float32
    hist = _hist_kernel(NP, nrows // NTILES)(dstp)
    xw1 = pl.pallas_call(
        _mm_body, out_shape=jax.ShapeDtypeStruct((NP, HID), f32))(xp, W1)
    y1, dinv = pl.pallas_call(
        _scale_body,
        out_shape=[jax.ShapeDtypeStruct((2, NP, HID // 2), f32),
                   jax.ShapeDtypeStruct((NP,), f32)])(xw1, hist)
    agg1 = _scatter_kernel(NP, HID // 2, nrows // NSUB, col_split=True)(
        y1, srcp, dstp)
    y2 = pl.pallas_call(
        _mid_body, out_shape=jax.ShapeDtypeStruct((NP, C), f32))(
        agg1, y1, dinv, b1, W2)
    agg2 = _scatter_kernel(NP, C, nrows // NTILES, col_split=False)(
        y2, srcp, dstp)
    out = pl.pallas_call(
        _fin_body, out_shape=jax.ShapeDtypeStruct((NP, C), f32))(
        agg2, y2, dinv, b2)
    return out[:N]


# speedup vs baseline: 7.8858x; 7.8858x over previous
import dataclasses
import functools

import jax
import jax.numpy as jnp
from jax import lax
from jax.experimental import pallas as pl
from jax.experimental.pallas import tpu as pltpu
from jax.experimental.pallas import tpu_sc as plsc

LANES = 16
BATCH = 128
NCORES = 2
NSUB = 16
NTILES = NCORES * NSUB


def _hist_kernel(NP, rows_per_tile):
    mesh = plsc.VectorSubcoreMesh(core_axis_name="c", subcore_axis_name="s")
    cp = pltpu.CompilerParams()
    if "needs_layout_passes" in pltpu.CompilerParams.__dataclass_fields__:
        cp = dataclasses.replace(cp, needs_layout_passes=False)

    @functools.partial(
        pl.kernel,
        out_type=jax.ShapeDtypeStruct((NTILES, NP), jnp.float32),
        mesh=mesh,
        compiler_params=cp,
        scratch_types=[
            pltpu.VMEM((rows_per_tile, BATCH), jnp.int32),
            pltpu.VMEM((NP,), jnp.float32),
        ],
    )
    def k(dst_hbm, out_hbm, dst_v, hist_v):
        c = lax.axis_index("c")
        s = lax.axis_index("s")
        wid = c * NSUB + s

        @pl.loop(0, NP // LANES)
        def _(i):
            hist_v[pl.ds(i * LANES, LANES)] = jnp.zeros((LANES,), jnp.float32)

        pltpu.sync_copy(dst_hbm.at[pl.ds(wid * rows_per_tile, rows_per_tile)],
                        dst_v)
        ones = jnp.ones((LANES,), jnp.float32)

        @pl.loop(0, rows_per_tile)
        def _(j):
            @pl.loop(0, BATCH // LANES)
            def _(g):
                idx = dst_v[j, pl.ds(g * LANES, LANES)]
                plsc.addupdate_scatter(hist_v, [idx], ones)

        pltpu.sync_copy(hist_v, out_hbm.at[wid])

    return k


def _scatter_kernel(NP, D, rows_per_tile, col_split):
    mesh = plsc.VectorSubcoreMesh(core_axis_name="c", subcore_axis_name="s")
    ST = NP // NSUB

    @functools.partial(
        pl.kernel,
        out_type=jax.ShapeDtypeStruct((NCORES, NP, D), jnp.float32),
        mesh=mesh,
        scratch_types=[
            pltpu.VMEM((rows_per_tile, BATCH), jnp.int32),
            pltpu.VMEM((rows_per_tile, BATCH), jnp.int32),
            pltpu.VMEM((BATCH, D), jnp.float32),
            pltpu.VMEM_SHARED((NP, D), jnp.float32),
        ],
    )
    def k(y_hbm, src_hbm, dst_hbm, out_hbm, src_v, dst_v, rows_v, agg_sh):
        c = lax.axis_index("c")
        s = lax.axis_index("s")
        base = s * ST

        @pl.loop(0, BATCH)
        def _(i):
            @pl.loop(0, D // LANES)
            def _(kk):
                rows_v[i, pl.ds(kk * LANES, LANES)] = (
                    jnp.zeros((LANES,), jnp.float32))

        for r0 in range(0, ST, BATCH):
            sz = min(BATCH, ST - r0)
            pltpu.sync_copy(rows_v.at[pl.ds(0, sz)],
                            agg_sh.at[pl.ds(base + r0, sz)])
        plsc.subcore_barrier()

        if col_split:
            row0 = s * rows_per_tile
            table = y_hbm.at[c]
        else:
            row0 = (c * NSUB + s) * rows_per_tile
            table = y_hbm

        pltpu.sync_copy(src_hbm.at[pl.ds(row0, rows_per_tile)], src_v)
        pltpu.sync_copy(dst_hbm.at[pl.ds(row0, rows_per_tile)], dst_v)

        @pl.loop(0, rows_per_tile)
        def _(j):
            pltpu.sync_copy(table.at[src_v.at[j]], rows_v)
            pltpu.sync_copy(rows_v, agg_sh.at[dst_v.at[j]], add=True)

        plsc.subcore_barrier()
        pltpu.sync_copy(agg_sh.at[pl.ds(base, ST)],
                        out_hbm.at[c].at[pl.ds(base, ST)])

    return k


def _mm_body(x_ref, w_ref, o_ref):
    o_ref[...] = jnp.dot(x_ref[...], w_ref[...],
                         preferred_element_type=jnp.float32)


def _scale_body(xw_ref, hist_ref, y_ref, dinv_ref):
    deg = jnp.sum(hist_ref[...], axis=0) + 1.0
    dinv = lax.rsqrt(deg)
    dinv_ref[...] = dinv
    y = xw_ref[...] * dinv[:, None]
    h = y.shape[1] // 2
    y_ref[0] = y[:, :h]
    y_ref[1] = y[:, h:]


def _mid_body(agg_ref, y1_ref, dinv_ref, b1_ref, w2_ref, y2_ref):
    dinv = dinv_ref[...][:, None]
    h = w2_ref.shape[0] // 2
    h0 = jnp.maximum((agg_ref[0] + y1_ref[0]) * dinv + b1_ref[:h][None, :],
                     0.0)
    h1 = jnp.maximum((agg_ref[1] + y1_ref[1]) * dinv + b1_ref[h:][None, :],
                     0.0)
    y2 = jnp.dot(h0, w2_ref[:h], preferred_element_type=jnp.float32)
    y2 = y2 + jnp.dot(h1, w2_ref[h:], preferred_element_type=jnp.float32)
    y2_ref[...] = y2 * dinv


def _fin_body(agg_ref, y2_ref, dinv_ref, b2_ref, o_ref):
    dinv = dinv_ref[...][:, None]
    o_ref[...] = ((agg_ref[0] + agg_ref[1] + y2_ref[...]) * dinv
                  + b2_ref[...][None, :])


def kernel(x, edge_index, W1, b1, W2, b2):
    N, F = x.shape
    HID = W1.shape[1]
    C = W2.shape[1]
    E = edge_index.shape[1]

    NP = ((N + 1 + 127) // 128) * 128
    EP = ((E + BATCH * NTILES - 1) // (BATCH * NTILES)) * (BATCH * NTILES)
    pad = EP - E
    nrows = EP // BATCH

    srcp = jnp.concatenate(
        [edge_index[0], jnp.full((pad,), N, jnp.int32)]).reshape(nrows, BATCH)
    dstp = jnp.concatenate(
        [edge_index[1], jnp.full((pad,), N, jnp.int32)]).reshape(nrows, BATCH)
    xp = jnp.concatenate([x, jnp.zeros((NP - N, F), x.dtype)])

    f32 = jnp.float32
    hist = _hist_kernel(NP, nrows // NTILES)(dstp)
    xw1 = pl.pallas_call(
        _mm_body, out_shape=jax.ShapeDtypeStruct((NP, HID), f32))(xp, W1)
    y1, dinv = pl.pallas_call(
        _scale_body,
        out_shape=[jax.ShapeDtypeStruct((2, NP, HID // 2), f32),
                   jax.ShapeDtypeStruct((NP,), f32)])(xw1, hist)
    agg1 = _scatter_kernel(NP, HID // 2, nrows // NSUB, col_split=True)(
        y1, srcp, dstp)
    y2 = pl.pallas_call(
        _mid_body, out_shape=jax.ShapeDtypeStruct((NP, C), f32))(
        agg1, y1, dinv, b1, W2)
    agg2 = _scatter_kernel(NP, C, nrows // NTILES, col_split=False)(
        y2, srcp, dstp)
    out = pl.pallas_call(
        _fin_body, out_shape=jax.ShapeDtypeStruct((NP, C), f32))(
        agg2, y2, dinv, b2)
    return out[:N]


# speedup vs baseline: 9.2330x; 1.1708x over previous
import dataclasses
import functools

import jax
import jax.numpy as jnp
from jax import lax
from jax.experimental import pallas as pl
from jax.experimental.pallas import tpu as pltpu
from jax.experimental.pallas import tpu_sc as plsc

LANES = 16
BATCH = 128
NCORES = 2
NSUB = 16
NTILES = NCORES * NSUB


def _hist_kernel(NP, rows_per_tile):
    mesh = plsc.VectorSubcoreMesh(core_axis_name="c", subcore_axis_name="s")
    cp = pltpu.CompilerParams()
    if "needs_layout_passes" in pltpu.CompilerParams.__dataclass_fields__:
        cp = dataclasses.replace(cp, needs_layout_passes=False)

    @functools.partial(
        pl.kernel,
        out_type=jax.ShapeDtypeStruct((NTILES, NP), jnp.float32),
        mesh=mesh,
        compiler_params=cp,
        scratch_types=[
            pltpu.VMEM((rows_per_tile, BATCH), jnp.int32),
            pltpu.VMEM((NP,), jnp.float32),
        ],
    )
    def k(dst_hbm, out_hbm, dst_v, hist_v):
        c = lax.axis_index("c")
        s = lax.axis_index("s")
        wid = c * NSUB + s

        @pl.loop(0, NP // LANES)
        def _(i):
            hist_v[pl.ds(i * LANES, LANES)] = jnp.zeros((LANES,), jnp.float32)

        pltpu.sync_copy(dst_hbm.at[pl.ds(wid * rows_per_tile, rows_per_tile)],
                        dst_v)
        ones = jnp.ones((LANES,), jnp.float32)

        @pl.loop(0, rows_per_tile)
        def _(j):
            @pl.loop(0, BATCH // LANES)
            def _(g):
                idx = dst_v[j, pl.ds(g * LANES, LANES)]
                plsc.addupdate_scatter(hist_v, [idx], ones)

        pltpu.sync_copy(hist_v, out_hbm.at[wid])

    return k


def _scatter_kernel(NP, D, rows_per_tile, col_split):
    mesh = plsc.VectorSubcoreMesh(core_axis_name="c", subcore_axis_name="s")
    ST = NP // NSUB
    win = rows_per_tile
    while win > 40:
        win //= 2
    assert win % 8 == 0 and win % 2 == 0

    @functools.partial(
        pl.kernel,
        out_type=jax.ShapeDtypeStruct((NCORES, NP, D), jnp.float32),
        mesh=mesh,
        scratch_types=[
            pltpu.VMEM((win, BATCH), jnp.int32),
            pltpu.VMEM((win, BATCH), jnp.int32),
            pltpu.VMEM((BATCH, D), jnp.float32),
            pltpu.VMEM((BATCH, D), jnp.float32),
            pltpu.VMEM_SHARED((NP, D), jnp.float32),
            pltpu.SemaphoreType.DMA,
            pltpu.SemaphoreType.DMA,
        ],
    )
    def k(y_hbm, src_hbm, dst_hbm, out_hbm, src_v, dst_v, buf0, buf1,
          agg_sh, sem0, sem1):
        c = lax.axis_index("c")
        s = lax.axis_index("s")
        base = s * ST

        @pl.loop(0, BATCH)
        def _(i):
            @pl.loop(0, D // LANES)
            def _(kk):
                buf0[i, pl.ds(kk * LANES, LANES)] = (
                    jnp.zeros((LANES,), jnp.float32))

        for r0 in range(0, ST, BATCH):
            sz = min(BATCH, ST - r0)
            pltpu.sync_copy(buf0.at[pl.ds(0, sz)],
                            agg_sh.at[pl.ds(base + r0, sz)])
        plsc.subcore_barrier()

        if col_split:
            row0 = s * rows_per_tile
            table = y_hbm.at[c]
        else:
            row0 = (c * NSUB + s) * rows_per_tile
            table = y_hbm

        nh = win // 2
        for half in range(rows_per_tile // win):
            r0h = row0 + half * win
            pltpu.sync_copy(src_hbm.at[pl.ds(r0h, win)], src_v)
            pltpu.sync_copy(dst_hbm.at[pl.ds(r0h, win)], dst_v)
            pltpu.async_copy(table.at[src_v.at[0]], buf0, sem0)
            pltpu.async_copy(table.at[src_v.at[1]], buf1, sem1)

            @pl.loop(0, nh)
            def _(jj):
                j0 = 2 * jj
                pltpu.make_async_copy(
                    table.at[src_v.at[j0]], buf0, sem0).wait()
                pltpu.sync_copy(buf0, agg_sh.at[dst_v.at[j0]], add=True)

                @pl.when(jj < nh - 1)
                def _():
                    pltpu.async_copy(table.at[src_v.at[j0 + 2]], buf0, sem0)

                pltpu.make_async_copy(
                    table.at[src_v.at[j0 + 1]], buf1, sem1).wait()
                pltpu.sync_copy(buf1, agg_sh.at[dst_v.at[j0 + 1]], add=True)

                @pl.when(jj < nh - 1)
                def _():
                    pltpu.async_copy(table.at[src_v.at[j0 + 3]], buf1, sem1)

        plsc.subcore_barrier()
        pltpu.sync_copy(agg_sh.at[pl.ds(base, ST)],
                        out_hbm.at[c].at[pl.ds(base, ST)])

    return k


def _mm_body(x_ref, w_ref, o_ref):
    o_ref[...] = jnp.dot(x_ref[...], w_ref[...],
                         preferred_element_type=jnp.float32)


def _scale_body(xw_ref, hist_ref, y_ref, dinv_ref):
    deg = jnp.sum(hist_ref[...], axis=0) + 1.0
    dinv = lax.rsqrt(deg)
    dinv_ref[...] = dinv
    y = xw_ref[...] * dinv[:, None]
    h = y.shape[1] // 2
    y_ref[0] = y[:, :h]
    y_ref[1] = y[:, h:]


def _mid_body(agg_ref, y1_ref, dinv_ref, b1_ref, w2_ref, y2_ref):
    dinv = dinv_ref[...][:, None]
    h = w2_ref.shape[0] // 2
    h0 = jnp.maximum((agg_ref[0] + y1_ref[0]) * dinv + b1_ref[:h][None, :],
                     0.0)
    h1 = jnp.maximum((agg_ref[1] + y1_ref[1]) * dinv + b1_ref[h:][None, :],
                     0.0)
    y2 = jnp.dot(h0, w2_ref[:h], preferred_element_type=jnp.float32)
    y2 = y2 + jnp.dot(h1, w2_ref[h:], preferred_element_type=jnp.float32)
    y2_ref[...] = y2 * dinv


def _fin_body(agg_ref, y2_ref, dinv_ref, b2_ref, o_ref):
    dinv = dinv_ref[...][:, None]
    o_ref[...] = ((agg_ref[0] + agg_ref[1] + y2_ref[...]) * dinv
                  + b2_ref[...][None, :])


def kernel(x, edge_index, W1, b1, W2, b2):
    N, F = x.shape
    HID = W1.shape[1]
    C = W2.shape[1]
    E = edge_index.shape[1]

    NP = ((N + 1 + 127) // 128) * 128
    EP = ((E + BATCH * NTILES - 1) // (BATCH * NTILES)) * (BATCH * NTILES)
    pad = EP - E
    nrows = EP // BATCH

    srcp = jnp.concatenate(
        [edge_index[0], jnp.full((pad,), N, jnp.int32)]).reshape(nrows, BATCH)
    dstp = jnp.concatenate(
        [edge_index[1], jnp.full((pad,), N, jnp.int32)]).reshape(nrows, BATCH)
    xp = jnp.concatenate([x, jnp.zeros((NP - N, F), x.dtype)])

    f32 = jnp.float32
    hist = _hist_kernel(NP, nrows // NTILES)(dstp)
    xw1 = pl.pallas_call(
        _mm_body, out_shape=jax.ShapeDtypeStruct((NP, HID), f32))(xp, W1)
    y1, dinv = pl.pallas_call(
        _scale_body,
        out_shape=[jax.ShapeDtypeStruct((2, NP, HID // 2), f32),
                   jax.ShapeDtypeStruct((NP,), f32)])(xw1, hist)
    agg1 = _scatter_kernel(NP, HID // 2, nrows // NSUB, col_split=True)(
        y1, srcp, dstp)
    y2 = pl.pallas_call(
        _mid_body, out_shape=jax.ShapeDtypeStruct((NP, C), f32))(
        agg1, y1, dinv, b1, W2)
    agg2 = _scatter_kernel(NP, C, nrows // NTILES, col_split=False)(
        y2, srcp, dstp)
    out = pl.pallas_call(
        _fin_body, out_shape=jax.ShapeDtypeStruct((NP, C), f32))(
        agg2, y2, dinv, b2)
    return out[:N]
